# Initial kernel scaffold; baseline (speedup 1.0000x reference)
import jax
import jax.numpy as jnp
from jax.experimental import pallas as pl


def kernel(x, edge_index, W1, att_src1, att_dst1, bias1, a1, W2, att_src2, att_dst2, bias2, a2):
    raise NotImplementedError("write your pallas kernel here")


# speedup vs baseline: 18.2557x; 18.2557x over previous
import functools

import jax
import jax.numpy as jnp
from jax import lax
from jax.experimental import pallas as pl
from jax.experimental.pallas import tpu as pltpu
from jax.experimental.pallas import tpu_sc as plsc

N = 10000
E = 320000
D = 128
DH = 64
NPAD = 10240
NB = 158
EPW = NB * 128
EPAD = 16 * EPW
BNEG = -1e9


def _head_body(x_ref, w_ref, asrc_ref, adst_ref, h_ref, as_ref, ad_ref):
    h = jnp.dot(x_ref[...], w_ref[...], preferred_element_type=jnp.float32)
    h_ref[0] = h[:, :DH]
    h_ref[1] = h[:, DH:]
    rid = pl.program_id(0) * 128 + lax.broadcasted_iota(jnp.int32, (128, 1), 0)
    valid = rid < N
    a_s = jnp.sum(h * asrc_ref[...], axis=1, keepdims=True)
    a_d = jnp.sum(h * adst_ref[...], axis=1, keepdims=True)
    as_ref[...] = jnp.where(valid, a_s, BNEG)
    ad_ref[...] = jnp.where(valid, a_d, BNEG)


def _head(x_pad, W, att_src, att_dst):
    return pl.pallas_call(
        _head_body,
        grid=(NPAD // 128,),
        in_specs=[
            pl.BlockSpec((128, D), lambda r: (r, 0)),
            pl.BlockSpec((D, D), lambda r: (0, 0)),
            pl.BlockSpec((1, D), lambda r: (0, 0)),
            pl.BlockSpec((1, D), lambda r: (0, 0)),
        ],
        out_specs=[
            pl.BlockSpec((2, 128, DH), lambda r: (0, r, 0)),
            pl.BlockSpec((128, 1), lambda r: (r, 0)),
            pl.BlockSpec((128, 1), lambda r: (r, 0)),
        ],
        out_shape=[
            jax.ShapeDtypeStruct((2, NPAD, DH), jnp.float32),
            jax.ShapeDtypeStruct((NPAD, 1), jnp.float32),
            jax.ShapeDtypeStruct((NPAD, 1), jnp.float32),
        ],
    )(x_pad, W, att_src, att_dst)


def _edge_body(src_hbm, dst_hbm, as_hbm, ad_hbm, h_hbm, num_out, den_out,
               src_v, dst_v, asv, adv, s_buf, rows, acc_num, acc_den):
    c = lax.axis_index("c")
    s = lax.axis_index("s")
    zeros16 = jnp.zeros((16,), jnp.float32)

    def zrow(i, _):
        for k in range(DH // 16):
            rows[i, pl.ds(k * 16, 16)] = zeros16
        return 0
    lax.fori_loop(0, 128, zrow, 0)
    for k in range(8):
        s_buf[pl.ds(k * 16, 16)] = zeros16
    base = s * 640
    for i in range(5):
        pltpu.sync_copy(rows, acc_num.at[pl.ds(base + i * 128, 128)])
        pltpu.sync_copy(s_buf, acc_den.at[pl.ds(base + i * 128, 128)])

    pltpu.sync_copy(as_hbm, asv)
    pltpu.sync_copy(ad_hbm, adv)
    pltpu.sync_copy(src_hbm.at[s], src_v)
    pltpu.sync_copy(dst_hbm.at[s], dst_v)
    plsc.subcore_barrier()

    def batch(b, _):
        for g in range(8):
            si = src_v[b, pl.ds(g * 16, 16)]
            di = dst_v[b, pl.ds(g * 16, 16)]
            t = plsc.load_gather(asv, [si]) + plsc.load_gather(adv, [di])
            t = jnp.where(t >= 0.0, t, 0.2 * t)
            s_buf[pl.ds(g * 16, 16)] = jnp.exp(t)
        pltpu.sync_copy(h_hbm.at[c].at[src_v.at[b]], rows)

        def scale(j, _):
            sv = plsc.load_gather(s_buf, [jnp.full((16,), j, jnp.int32)])
            for k in range(DH // 16):
                rows[j, pl.ds(k * 16, 16)] = rows[j, pl.ds(k * 16, 16)] * sv
            return 0
        lax.fori_loop(0, 128, scale, 0)

        pltpu.sync_copy(rows, acc_num.at[dst_v.at[b]], add=True)
        pltpu.sync_copy(s_buf, acc_den.at[dst_v.at[b]], add=True)
        return 0
    lax.fori_loop(0, NB, batch, 0)

    plsc.subcore_barrier()
    pltpu.sync_copy(acc_num.at[pl.ds(base, 640)],
                    num_out.at[c, pl.ds(base, 640)])
    pltpu.sync_copy(acc_den.at[pl.ds(base, 640)],
                    den_out.at[c, pl.ds(base, 640)])


_edge_call = functools.partial(
    pl.kernel,
    out_type=(jax.ShapeDtypeStruct((2, NPAD, DH), jnp.float32),
              jax.ShapeDtypeStruct((2, NPAD), jnp.float32)),
    mesh=plsc.VectorSubcoreMesh(core_axis_name="c", subcore_axis_name="s"),
    compiler_params=pltpu.CompilerParams(
        use_tc_tiling_on_sc=False, needs_layout_passes=False),
    scratch_types=[
        pltpu.VMEM((NB, 128), jnp.int32),
        pltpu.VMEM((NB, 128), jnp.int32),
        pltpu.VMEM((NPAD,), jnp.float32),
        pltpu.VMEM((NPAD,), jnp.float32),
        pltpu.VMEM((128,), jnp.float32),
        pltpu.VMEM((128, DH), jnp.float32),
        pltpu.VMEM_SHARED((NPAD, DH), jnp.float32),
        pltpu.VMEM_SHARED((NPAD,), jnp.float32),
    ],
)(_edge_body)


def _tail_body(num_ref, den_ref, h_ref, asrc_ref, adst_ref, b_ref, a_ref,
               o_ref):
    h = jnp.concatenate([h_ref[0], h_ref[1]], axis=1)
    t = (jnp.sum(h * asrc_ref[...], axis=1, keepdims=True)
         + jnp.sum(h * adst_ref[...], axis=1, keepdims=True))
    w_self = jnp.exp(jnp.where(t >= 0.0, t, 0.2 * t))
    num = jnp.concatenate([num_ref[0], num_ref[1]], axis=1) + w_self * h
    den = den_ref[0] + w_self + 1e-16
    out = num / den + b_ref[...]
    a = a_ref[0, 0]
    o_ref[...] = jnp.where(out >= 0.0, out, a * out)


def _tail(num, den, h3, att_src, att_dst, bias, a):
    return pl.pallas_call(
        _tail_body,
        grid=(NPAD // 128,),
        in_specs=[
            pl.BlockSpec((2, 128, DH), lambda r: (0, r, 0)),
            pl.BlockSpec((2, 128, 1), lambda r: (0, r, 0)),
            pl.BlockSpec((2, 128, DH), lambda r: (0, r, 0)),
            pl.BlockSpec((1, D), lambda r: (0, 0)),
            pl.BlockSpec((1, D), lambda r: (0, 0)),
            pl.BlockSpec((1, D), lambda r: (0, 0)),
            pl.BlockSpec((1, 1), lambda r: (0, 0)),
        ],
        out_specs=pl.BlockSpec((128, D), lambda r: (r, 0)),
        out_shape=jax.ShapeDtypeStruct((NPAD, D), jnp.float32),
    )(num, den, h3, att_src, att_dst, bias, a)


def _layer(x_pad, src_g, dst_g, W, att_src, att_dst, bias, a):
    asr = att_src.reshape(1, D)
    adr = att_dst.reshape(1, D)
    h3, a_s, a_d = _head(x_pad, W, asr, adr)
    num, den = _edge_call(src_g, dst_g, a_s.reshape(NPAD), a_d.reshape(NPAD),
                          h3)
    return _tail(num, den.reshape(2, NPAD, 1), h3, asr, adr,
                 bias.reshape(1, D), a.reshape(1, 1))


def kernel(x, edge_index, W1, att_src1, att_dst1, bias1, a1,
           W2, att_src2, att_dst2, bias2, a2):
    pad_idx = N + (jnp.arange(EPAD - E, dtype=jnp.int32) % 16)
    src_g = jnp.concatenate([edge_index[0], pad_idx]).reshape(16, NB, 128)
    dst_g = jnp.concatenate([edge_index[1], pad_idx]).reshape(16, NB, 128)
    x_pad = jnp.pad(x, ((0, NPAD - N), (0, 0)))
    x2 = _layer(x_pad, src_g, dst_g, W1, att_src1, att_dst1, bias1, a1)
    out = _layer(x2, src_g, dst_g, W2, att_src2, att_dst2, bias2, a2)
    return out[:N]
